# baseline (speedup 1.0000x reference)
import functools

import jax
import jax.numpy as jnp
from jax import lax
from jax.experimental import pallas as pl
from jax.experimental.pallas import tpu as pltpu
from jax.experimental.pallas import tpu_sc as plsc

N_EMB = 1000000
EMB_DIM = 32
SCALE = 10.0
LANES = 16

NUM_CORES = 2
NUM_SUBCORES = 16
NW = NUM_CORES * NUM_SUBCORES

XROWS = 16384
XCOLS = 100
IPW = XROWS // NW
NGRP = IPW // LANES

_mesh = plsc.VectorSubcoreMesh(core_axis_name="c", subcore_axis_name="s")


@functools.partial(
    pl.kernel,
    mesh=_mesh,
    out_type=jax.ShapeDtypeStruct((XCOLS, EMB_DIM, XROWS), jnp.float32),
    scratch_types=[
        pltpu.VMEM((IPW,), jnp.int32),
        pltpu.VMEM((IPW, EMB_DIM), jnp.float32),
        pltpu.VMEM((EMB_DIM, IPW), jnp.float32),
        pltpu.SemaphoreType.DMA,
    ],
    compiler_params=pltpu.CompilerParams(
        use_tc_tiling_on_sc=False, needs_layout_passes=False
    ),
)
def _scaled_gather(xt_hbm, tab_hbm, out_hbm, idx_v, g_v, o_v, sem):
    wid = lax.axis_index("s") * NUM_CORES + lax.axis_index("c")
    i0 = wid * IPW
    iota = lax.iota(jnp.int32, LANES)

    def col_body(j, carry):
        pltpu.sync_copy(xt_hbm.at[j, pl.ds(i0, IPW)], idx_v)
        pltpu.async_copy(tab_hbm.at[idx_v], g_v, sem).wait()

        def grp_body(grp, c1):
            rows = iota + grp * LANES
            for k in range(EMB_DIM):
                cols = jnp.full((LANES,), k, jnp.int32)
                vals = plsc.load_gather(g_v, [rows, cols])
                o_v[k, pl.ds(grp * LANES, LANES)] = vals * SCALE
            return c1

        lax.fori_loop(0, NGRP, grp_body, 0)
        pltpu.sync_copy(o_v, out_hbm.at[j, :, pl.ds(i0, IPW)])
        return carry

    lax.fori_loop(0, XCOLS, col_body, 0)


def kernel(x, table):
    out = _scaled_gather(x.T, table)
    return out.transpose(2, 0, 1)


# speedup vs baseline: 1.5359x; 1.5359x over previous
import functools

import jax
import jax.numpy as jnp
from jax import lax
from jax.experimental import pallas as pl
from jax.experimental.pallas import tpu as pltpu
from jax.experimental.pallas import tpu_sc as plsc

N_EMB = 1000000
EMB_DIM = 32
SCALE = 10.0
LANES = 16

NUM_CORES = 2
NUM_SUBCORES = 16
NW = NUM_CORES * NUM_SUBCORES

XROWS = 16384
XCOLS = 100
IPW = XROWS // NW
NGRP = IPW // LANES

_mesh = plsc.VectorSubcoreMesh(core_axis_name="c", subcore_axis_name="s")


@functools.partial(
    pl.kernel,
    mesh=_mesh,
    out_type=jax.ShapeDtypeStruct((XCOLS, EMB_DIM, XROWS), jnp.float32),
    scratch_types=[
        pltpu.VMEM((IPW,), jnp.int32),
        pltpu.VMEM((IPW, EMB_DIM), jnp.float32),
        pltpu.VMEM((EMB_DIM, IPW + 1), jnp.float32),
        pltpu.SemaphoreType.DMA,
    ],
    compiler_params=pltpu.CompilerParams(
        use_tc_tiling_on_sc=False, needs_layout_passes=False
    ),
)
def _scaled_gather(xt_hbm, tab_hbm, out_hbm, idx_v, g_v, o_v, sem):
    wid = lax.axis_index("s") * NUM_CORES + lax.axis_index("c")
    i0 = wid * IPW
    iota = lax.iota(jnp.int32, LANES)

    def col_body(j, carry):
        pltpu.sync_copy(xt_hbm.at[j, pl.ds(i0, IPW)], idx_v)
        pltpu.async_copy(tab_hbm.at[idx_v], g_v, sem).wait()

        def row_body(r, c1):
            col = jnp.broadcast_to(r, (LANES,)).astype(jnp.int32)
            for half in range(EMB_DIM // LANES):
                vals = g_v[r, pl.ds(half * LANES, LANES)] * SCALE
                plsc.store_scatter(o_v, [iota + half * LANES, col], vals)
            return c1

        lax.fori_loop(0, IPW, row_body, 0)
        pltpu.sync_copy(o_v.at[:, pl.ds(0, IPW)],
                        out_hbm.at[j, :, pl.ds(i0, IPW)])
        return carry

    lax.fori_loop(0, XCOLS, col_body, 0)


def kernel(x, table):
    out = _scaled_gather(x.T, table)
    return out.transpose(2, 0, 1)


# speedup vs baseline: 2.1733x; 1.4150x over previous
import functools

import jax
import jax.numpy as jnp
from jax import lax
from jax.experimental import pallas as pl
from jax.experimental.pallas import tpu as pltpu
from jax.experimental.pallas import tpu_sc as plsc

N_EMB = 1000000
EMB_DIM = 32
SCALE = 10.0
LANES = 16

NUM_CORES = 2
NUM_SUBCORES = 16
NW = NUM_CORES * NUM_SUBCORES

XROWS = 16384
XCOLS = 100
IPW = XROWS // NW
NGRP = IPW // LANES

_mesh = plsc.VectorSubcoreMesh(core_axis_name="c", subcore_axis_name="s")


@functools.partial(
    pl.kernel,
    mesh=_mesh,
    out_type=jax.ShapeDtypeStruct((XCOLS, EMB_DIM, XROWS), jnp.float32),
    scratch_types=[
        pltpu.VMEM((IPW,), jnp.int32),
        pltpu.VMEM((IPW, EMB_DIM), jnp.float32),
        pltpu.VMEM((EMB_DIM, IPW + 1), jnp.float32),
        pltpu.SemaphoreType.DMA,
    ],
    compiler_params=pltpu.CompilerParams(
        use_tc_tiling_on_sc=False, needs_layout_passes=False
    ),
)
def _scaled_gather(xt_hbm, tab_hbm, out_hbm, idx_v, g_v, o_v, sem):
    wid = lax.axis_index("s") * NUM_CORES + lax.axis_index("c")
    i0 = wid * IPW
    iota = lax.iota(jnp.int32, LANES)

    def col_body(j, carry):
        pltpu.sync_copy(xt_hbm.at[j, pl.ds(i0, IPW)], idx_v)
        pltpu.async_copy(tab_hbm.at[idx_v], g_v, sem).wait()

        @plsc.parallel_loop(0, IPW, unroll=8)
        def row_body(r):
            col = jnp.broadcast_to(r, (LANES,)).astype(jnp.int32)
            for half in range(EMB_DIM // LANES):
                vals = g_v[r, pl.ds(half * LANES, LANES)] * SCALE
                plsc.store_scatter(o_v, [iota + half * LANES, col], vals)
        pltpu.sync_copy(o_v.at[:, pl.ds(0, IPW)],
                        out_hbm.at[j, :, pl.ds(i0, IPW)])
        return carry

    lax.fori_loop(0, XCOLS, col_body, 0)


def kernel(x, table):
    out = _scaled_gather(x.T, table)
    return out.transpose(2, 0, 1)


# speedup vs baseline: 2.5843x; 1.1891x over previous
import functools

import jax
import jax.numpy as jnp
from jax import lax
from jax.experimental import pallas as pl
from jax.experimental.pallas import tpu as pltpu
from jax.experimental.pallas import tpu_sc as plsc

N_EMB = 1000000
EMB_DIM = 32
SCALE = 10.0
LANES = 16

NUM_CORES = 2
NUM_SUBCORES = 16
NW = NUM_CORES * NUM_SUBCORES

XROWS = 16384
XCOLS = 100
IPW = XROWS // NW
NGRP = IPW // LANES

_mesh = plsc.VectorSubcoreMesh(core_axis_name="c", subcore_axis_name="s")


@functools.partial(
    pl.kernel,
    mesh=_mesh,
    out_type=jax.ShapeDtypeStruct((XCOLS, EMB_DIM, XROWS), jnp.float32),
    scratch_types=[
        pltpu.VMEM((IPW,), jnp.int32),
        pltpu.VMEM((IPW,), jnp.int32),
        pltpu.VMEM((IPW, EMB_DIM), jnp.float32),
        pltpu.VMEM((IPW, EMB_DIM), jnp.float32),
        pltpu.VMEM((EMB_DIM, IPW + 1), jnp.float32),
        pltpu.VMEM((EMB_DIM, IPW + 1), jnp.float32),
        pltpu.SemaphoreType.DMA,
        pltpu.SemaphoreType.DMA,
        pltpu.SemaphoreType.DMA,
        pltpu.SemaphoreType.DMA,
    ],
    compiler_params=pltpu.CompilerParams(
        use_tc_tiling_on_sc=False, needs_layout_passes=False
    ),
)
def _scaled_gather(xt_hbm, tab_hbm, out_hbm, idx0, idx1, g0, g1, o0, o1,
                   sg0, sg1, so0, so1):
    wid = lax.axis_index("s") * NUM_CORES + lax.axis_index("c")
    i0 = wid * IPW
    iota = lax.iota(jnp.int32, LANES)
    idx = (idx0, idx1)
    gv = (g0, g1)
    ov = (o0, o1)
    sg = (sg0, sg1)
    so = (so0, so1)

    def stage(j, b):
        pltpu.sync_copy(xt_hbm.at[j, pl.ds(i0, IPW)], idx[b])
        pltpu.async_copy(tab_hbm.at[idx[b]], gv[b], sg[b])

    def transpose(b):
        @plsc.parallel_loop(0, IPW, unroll=8)
        def row_body(r):
            col = jnp.broadcast_to(r, (LANES,)).astype(jnp.int32)
            for half in range(EMB_DIM // LANES):
                vals = gv[b][r, pl.ds(half * LANES, LANES)] * SCALE
                plsc.store_scatter(ov[b], [iota + half * LANES, col], vals)

    def out_slice(j):
        return out_hbm.at[j, :, pl.ds(i0, IPW)]

    stage(0, 0)

    def pair_body(gp, carry):
        for b in range(2):
            j = 2 * gp + b
            nb = 1 - b

            @pl.when(j + 1 < XCOLS)
            def _():
                stage(j + 1, nb)

            pltpu.make_async_copy(tab_hbm.at[idx[b]], gv[b], sg[b]).wait()

            @pl.when(j >= 2)
            def _():
                pltpu.make_async_copy(
                    ov[b].at[:, pl.ds(0, IPW)], out_slice(j), so[b]
                ).wait()

            transpose(b)
            pltpu.async_copy(ov[b].at[:, pl.ds(0, IPW)], out_slice(j), so[b])
        return carry

    lax.fori_loop(0, XCOLS // 2, pair_body, 0)
    for b in range(2):
        pltpu.make_async_copy(
            ov[b].at[:, pl.ds(0, IPW)], out_slice(XCOLS - 2 + b), so[b]
        ).wait()


def kernel(x, table):
    out = _scaled_gather(x.T, table)
    return out.transpose(2, 0, 1)
